# baseline (speedup 1.0000x reference)
import functools
import math

import jax
import jax.numpy as jnp
from jax import lax
from jax.experimental import pallas as pl
from jax.experimental.pallas import tpu as pltpu
from jax.experimental.pallas import tpu_sc as plsc

EMB = 64
SCALE = math.sqrt(EMB)
NC = 2
NS = 16
NW = NC * NS
L = 16
TAIL = 64
SG = 4


def _conv_body(v, tab_t_hbm, side_hbm, out_hbm, sbuf, obuf, g_sems, s_sems):
    wid = lax.axis_index("s") * NC + lax.axis_index("c")
    n_blocks = (v - TAIL) // 128
    per_tile = (n_blocks + NW - 1) // NW
    iota = lax.iota(jnp.int32, L)

    def stage(blk, bb):
        r0 = blk * 128
        for i in range(8):
            pltpu.async_copy(
                tab_t_hbm.at[pl.ds(8 * i, 8), pl.ds(r0, 128)],
                sbuf.at[bb, pl.ds(8 * i, 8)],
                g_sems.at[bb],
            )

    def wait_stage(blk, bb):
        r0 = blk * 128
        for i in range(8):
            pltpu.make_async_copy(
                tab_t_hbm.at[pl.ds(8 * i, 8), pl.ds(r0, 128)],
                sbuf.at[bb, pl.ds(8 * i, 8)],
                g_sems.at[bb],
            ).wait()

    krows = [lax.shift_right_logical(iota + 16 * g, 1) for g in range(8)]
    kcols = [lax.rem(iota + 16 * g, 2) * EMB for g in range(8)]

    def transpose_to(bb, src):
        @plsc.parallel_loop(0, EMB, unroll=2)
        def _(u):
            for g in range(8):
                val = src[u, pl.ds(g * L, L)] * SCALE
                plsc.store_scatter(obuf.at[bb], [krows[g], kcols[g] + u], val)

    def store(out_row, bb):
        pltpu.async_copy(
            obuf.at[bb, :, pl.ds(0, 128)],
            out_hbm.at[pl.ds(out_row, 64)],
            s_sems.at[bb],
        )

    def wait_store(out_row, bb):
        pltpu.make_async_copy(
            obuf.at[bb, :, pl.ds(0, 128)],
            out_hbm.at[pl.ds(out_row, 64)],
            s_sems.at[bb],
        ).wait()

    def blk_of(j):
        return j * NW + wid

    @pl.when(blk_of(0) < n_blocks)
    def _():
        stage(blk_of(0), 0)

    def j_body(j, carry):
        bb = lax.rem(j, 2)
        nb = lax.rem(j + 1, 2)

        @pl.when(jnp.logical_and(j >= 1, blk_of(j - 1) < n_blocks))
        def _():
            wait_store(blk_of(j - 1) * 64, nb)

        @pl.when(blk_of(j + 1) < n_blocks)
        def _():
            stage(blk_of(j + 1), nb)

        @pl.when(blk_of(j) < n_blocks)
        def _():
            wait_stage(blk_of(j), bb)
            transpose_to(bb, sbuf.at[bb])
            store(blk_of(j) * 64, bb)

        return carry

    lax.fori_loop(0, per_tile, j_body, 0)

    @pl.when(blk_of(per_tile - 1) < n_blocks)
    def _():
        wait_store(blk_of(per_tile - 1) * 64, (per_tile - 1) % 2)

    @pl.when(wid == 0)
    def _():
        pltpu.sync_copy(side_hbm.at[:, pl.ds(0, 128)], sbuf.at[0])

        @plsc.parallel_loop(0, 32, unroll=2)
        def _(k):
            for h in range(2):
                for q in range(4):
                    rowv = jnp.full((L,), 2 * k + h, jnp.int32)
                    colv = iota + (q * L)
                    val = plsc.load_gather(sbuf.at[0], [rowv, colv])
                    obuf[0, k, pl.ds(h * EMB + q * L, L)] = val * SCALE

        pltpu.sync_copy(obuf.at[0, pl.ds(0, 32), pl.ds(0, 128)],
                        out_hbm.at[pl.ds((v - TAIL) // 2, 32)])


def _emb_body(seq, tok_hbm, table_hbm, out_hbm,
              traw, idxf, gbuf, obuf, g_sems, s_sems):
    wid = lax.axis_index("s") * NC + lax.axis_index("c")
    b0 = wid * 128
    n_groups = seq // SG
    gn = SG * 128

    def stage_group(gr, bb):
        pltpu.sync_copy(tok_hbm.at[pl.ds(gr * SG, SG), pl.ds(b0, 128)],
                        traw.at[bb])

        @plsc.parallel_loop(0, gn // L, unroll=4)
        def _(i):
            r = i // (128 // L)
            sl = pl.ds((i % (128 // L)) * L, L)
            idxf[bb, pl.ds(i * L, L)] = traw[bb, r, sl]

    def start_gather(gr, bb):
        pltpu.async_copy(
            table_hbm.at[idxf.at[bb]],
            gbuf.at[bb],
            g_sems.at[bb],
        )

    def wait_gather(gr, bb):
        pltpu.make_async_copy(
            table_hbm.at[idxf.at[bb]],
            gbuf.at[bb],
            g_sems.at[bb],
        ).wait()

    def wait_store(s, ob):
        pltpu.make_async_copy(
            obuf.at[ob, :, pl.ds(0, 128)],
            out_hbm.at[s, :, pl.ds(b0, 128)],
            s_sems.at[ob],
        ).wait()

    stage_group(0, 0)
    start_gather(0, 0)
    iota = lax.iota(jnp.int32, L)

    def g_body(gr, carry):
        bb = lax.rem(gr, 2)
        nb = lax.rem(gr + 1, 2)

        @pl.when(gr + 1 < n_groups)
        def _():
            stage_group(gr + 1, nb)
            start_gather(gr + 1, nb)

        wait_gather(gr, bb)

        for s8 in range(SG):
            s = gr * SG + s8
            ob = lax.rem(s, 2)

            @pl.when(s >= 2)
            def _():
                wait_store(s - 2, ob)

            @plsc.parallel_loop(0, 128, unroll=2)
            def _(r):
                bcol = jnp.full((L,), r, jnp.int32)
                for j in range(EMB // L):
                    val = gbuf[bb, s8 * 128 + r, pl.ds(j * L, L)]
                    plsc.store_scatter(obuf.at[ob], [iota + j * L, bcol], val)

            pltpu.async_copy(
                obuf.at[ob, :, pl.ds(0, 128)],
                out_hbm.at[s, :, pl.ds(b0, 128)],
                s_sems.at[ob],
            )
        return carry

    lax.fori_loop(0, n_groups, g_body, 0)
    wait_store(seq - 2, lax.rem(seq - 2, 2))
    wait_store(seq - 1, lax.rem(seq - 1, 2))


def kernel(tokens, embedding_weight):
    b, s = tokens.shape
    v, e = embedding_weight.shape
    assert b == NW * 128 and e == EMB and s % (2 * SG) == 0

    mesh = plsc.VectorSubcoreMesh(core_axis_name="c", subcore_axis_name="s")

    table_t = embedding_weight.T
    side = jnp.pad(embedding_weight[v - TAIL:, :], ((0, 0), (0, 2 * e - EMB)))
    conv = pl.kernel(
        functools.partial(_conv_body, v),
        mesh=mesh,
        out_type=jax.ShapeDtypeStruct((v // 2, 2 * e), jnp.float32),
        scratch_types=[
            pltpu.VMEM((2, EMB, 128), jnp.float32),
            pltpu.VMEM((2, 64, 129), jnp.float32),
            pltpu.SemaphoreType.DMA((2,)),
            pltpu.SemaphoreType.DMA((2,)),
        ],
        compiler_params=pltpu.CompilerParams(
            use_tc_tiling_on_sc=True, needs_layout_passes=False),
    )
    table2 = conv(table_t, side)
    table3 = table2.reshape(v, e)

    tokens_t = tokens.T.astype(jnp.int32)
    run = pl.kernel(
        functools.partial(_emb_body, s),
        mesh=mesh,
        out_type=jax.ShapeDtypeStruct((s, EMB, b), jnp.float32),
        scratch_types=[
            pltpu.VMEM((2, SG, 128), jnp.int32),
            pltpu.VMEM((2, SG * 128), jnp.int32),
            pltpu.VMEM((2, SG * 128, EMB), jnp.float32),
            pltpu.VMEM((2, EMB, 129), jnp.float32),
            pltpu.SemaphoreType.DMA((2,)),
            pltpu.SemaphoreType.DMA((2,)),
        ],
        compiler_params=pltpu.CompilerParams(
            use_tc_tiling_on_sc=False, needs_layout_passes=False),
    )
    out = run(tokens_t, table3)
    return out.transpose(2, 0, 1)


# speedup vs baseline: 1.3677x; 1.3677x over previous
import functools
import math

import jax
import jax.numpy as jnp
from jax import lax
from jax.experimental import pallas as pl
from jax.experimental.pallas import tpu as pltpu
from jax.experimental.pallas import tpu_sc as plsc

EMB = 64
SCALE = math.sqrt(EMB)
NC = 2
NS = 16
NW = NC * NS
L = 16
SG = 4

def _emb_body(seq, tok_hbm, table_hbm, out_hbm,
              traw, idxf, gbuf, obuf, g_sems, s_sems):
    wid = lax.axis_index("s") * NC + lax.axis_index("c")
    b0 = wid * 128
    n_groups = seq // SG
    gn = SG * 128

    def stage_group(gr, bb):
        pltpu.sync_copy(tok_hbm.at[pl.ds(gr * SG, SG), pl.ds(b0, 128)],
                        traw.at[bb])

        @plsc.parallel_loop(0, gn // L, unroll=4)
        def _(i):
            r = i // (128 // L)
            sl = pl.ds((i % (128 // L)) * L, L)
            idxf[bb, pl.ds(i * L, L)] = traw[bb, r, sl]

    def start_gather(gr, bb):
        pltpu.async_copy(
            table_hbm.at[idxf.at[bb]],
            gbuf.at[bb],
            g_sems.at[bb],
        )

    def wait_gather(gr, bb):
        pltpu.make_async_copy(
            table_hbm.at[idxf.at[bb]],
            gbuf.at[bb],
            g_sems.at[bb],
        ).wait()

    def wait_store(s, ob):
        pltpu.make_async_copy(
            obuf.at[ob, :, pl.ds(0, 128)],
            out_hbm.at[s, :, pl.ds(b0, 128)],
            s_sems.at[ob],
        ).wait()

    stage_group(0, 0)
    start_gather(0, 0)
    iota = lax.iota(jnp.int32, L)

    def g_body(gr, carry):
        bb = lax.rem(gr, 2)
        nb = lax.rem(gr + 1, 2)

        @pl.when(gr + 1 < n_groups)
        def _():
            stage_group(gr + 1, nb)
            start_gather(gr + 1, nb)

        wait_gather(gr, bb)

        for s8 in range(SG):
            s = gr * SG + s8
            ob = lax.rem(s, 2)

            @pl.when(s >= 2)
            def _():
                wait_store(s - 2, ob)

            @plsc.parallel_loop(0, 128, unroll=2)
            def _(r):
                bcol = jnp.full((L,), r, jnp.int32)
                for j in range(EMB // L):
                    val = gbuf[bb, s8 * 128 + r, pl.ds(j * L, L)] * SCALE
                    plsc.store_scatter(obuf.at[ob], [iota + j * L, bcol], val)

            pltpu.async_copy(
                obuf.at[ob, :, pl.ds(0, 128)],
                out_hbm.at[s, :, pl.ds(b0, 128)],
                s_sems.at[ob],
            )
        return carry

    lax.fori_loop(0, n_groups, g_body, 0)
    wait_store(seq - 2, lax.rem(seq - 2, 2))
    wait_store(seq - 1, lax.rem(seq - 1, 2))


def kernel(tokens, embedding_weight):
    b, s = tokens.shape
    v, e = embedding_weight.shape
    assert b == NW * 128 and e == EMB and s % (2 * SG) == 0

    mesh = plsc.VectorSubcoreMesh(core_axis_name="c", subcore_axis_name="s")
    tokens_t = tokens.T.astype(jnp.int32)
    run = pl.kernel(
        functools.partial(_emb_body, s),
        mesh=mesh,
        out_type=jax.ShapeDtypeStruct((s, EMB, b), jnp.float32),
        scratch_types=[
            pltpu.VMEM((2, SG, 128), jnp.int32),
            pltpu.VMEM((2, SG * 128), jnp.int32),
            pltpu.VMEM((2, SG * 128, EMB), jnp.float32),
            pltpu.VMEM((2, EMB, 129), jnp.float32),
            pltpu.SemaphoreType.DMA((2,)),
            pltpu.SemaphoreType.DMA((2,)),
        ],
        compiler_params=pltpu.CompilerParams(
            use_tc_tiling_on_sc=False, needs_layout_passes=False),
    )
    out = run(tokens_t, embedding_weight)
    return out.transpose(2, 0, 1)
